# baseline (speedup 1.0000x reference)
import functools

import jax
import jax.numpy as jnp
from jax.experimental import pallas as pl
from jax.experimental.pallas import tpu as pltpu


def _roll(v, shift):
    return pltpu.roll(v, shift % v.shape[1], axis=1)


def _body(x_ref, w_ref, b_ref, pay_ref, o_ref, *, N, K):
    m = jax.lax.dot_general(
        x_ref[0], w_ref[...],
        dimension_numbers=(((0,), (1,)), ((), ())),
        preferred_element_type=jnp.float32,
    ) + b_ref[...]
    mx = jnp.max(m, axis=1, keepdims=True)
    e = jnp.exp(m - mx)
    s = jnp.sum(e, axis=1, keepdims=True)
    key = e / s

    TI = key.shape[0]
    iota = jax.lax.broadcasted_iota(jnp.int32, (TI, N), 1)
    idx = iota
    pay = pay_ref[0]

    kk = 2
    while kk <= N:
        asc = (iota & kk) != 0
        d = kk // 2
        while d >= 1:
            is_high = (iota & d) != 0
            key_p = jnp.where(is_high, _roll(key, d), _roll(key, -d))
            idx_p = jnp.where(is_high, _roll(idx, d), _roll(idx, -d))
            pay_p = jnp.where(is_high, _roll(pay, d), _roll(pay, -d))
            gt = (key_p > key) | ((key_p == key) & (idx_p < idx))
            take = gt == (is_high == asc)
            key = jnp.where(take, key_p, key)
            idx = jnp.where(take, idx_p, idx)
            pay = jnp.where(take, pay_p, pay)
            d //= 2
        kk *= 2

    o_ref[0] = pay[:, :K]


def kernel(x, W, b):
    B, L, D = x.shape
    N = D
    K = int((1.0 - 0.75) * L)
    TI = 256 if D % 256 == 0 else D
    grid = (B, D // TI)
    body = functools.partial(_body, N=N, K=K)
    out = pl.pallas_call(
        body,
        grid=grid,
        in_specs=[
            pl.BlockSpec((1, L, TI), lambda bb, i: (bb, 0, i)),
            pl.BlockSpec((D, L), lambda bb, i: (0, 0)),
            pl.BlockSpec((1, D), lambda bb, i: (0, 0)),
            pl.BlockSpec((1, TI, D), lambda bb, i: (bb, i, 0)),
        ],
        out_specs=pl.BlockSpec((1, TI, K), lambda bb, i: (bb, i, 0)),
        out_shape=jax.ShapeDtypeStruct((B, D, K), jnp.float32),
        compiler_params=pltpu.CompilerParams(
            dimension_semantics=("parallel", "parallel"),
        ),
    )(x, W, b.reshape(1, D), x)
    return jnp.transpose(out, (0, 2, 1))


# speedup vs baseline: 1.1492x; 1.1492x over previous
import functools

import jax
import jax.numpy as jnp
from jax.experimental import pallas as pl
from jax.experimental.pallas import tpu as pltpu


def _roll(v, shift):
    return pltpu.roll(v, shift % v.shape[1], axis=1)


def _body(x_ref, w_ref, b_ref, pay_ref, o_ref, *, N, K):
    m = jax.lax.dot_general(
        x_ref[0], w_ref[...],
        dimension_numbers=(((0,), (1,)), ((), ())),
        preferred_element_type=jnp.float32,
    ) + b_ref[...]
    mx = jnp.max(m, axis=1, keepdims=True)
    e = jnp.exp(m - mx)
    s = jnp.sum(e, axis=1, keepdims=True)
    key = e / s

    TI = key.shape[0]
    iota = jax.lax.broadcasted_iota(jnp.int32, (TI, N), 1)
    idx = iota
    pay = pay_ref[0]

    def ce(key, idx, pay, iota, d, asc):
        is_high = (iota & d) != 0
        key_p = jnp.where(is_high, _roll(key, d), _roll(key, -d))
        idx_p = jnp.where(is_high, _roll(idx, d), _roll(idx, -d))
        pay_p = jnp.where(is_high, _roll(pay, d), _roll(pay, -d))
        gt = (key_p > key) | ((key_p == key) & (idx_p < idx))
        take = gt == (is_high == asc)
        return (jnp.where(take, key_p, key), jnp.where(take, idx_p, idx),
                jnp.where(take, pay_p, pay))

    kk = 2
    while kk <= K:
        asc = (iota & kk) != 0
        d = kk // 2
        while d >= 1:
            key, idx, pay = ce(key, idx, pay, iota, d, asc)
            d //= 2
        kk *= 2
    key, idx, pay = ce(key, idx, pay, iota, K, (iota & (2 * K)) != 0)
    key = jnp.concatenate([key[:, 0:K], key[:, 3 * K:4 * K]], axis=1)
    idx = jnp.concatenate([idx[:, 0:K], idx[:, 3 * K:4 * K]], axis=1)
    pay = jnp.concatenate([pay[:, 0:K], pay[:, 3 * K:4 * K]], axis=1)
    iota2 = jax.lax.broadcasted_iota(jnp.int32, (TI, 2 * K), 1)
    asc = (iota2 & K) != 0
    d = K // 2
    while d >= 1:
        key, idx, pay = ce(key, idx, pay, iota2, d, asc)
        d //= 2
    key, idx, pay = ce(key, idx, pay, iota2, K, (iota2 & (2 * K)) != 0)
    key = key[:, 0:K]
    idx = idx[:, 0:K]
    pay = pay[:, 0:K]
    iota3 = jax.lax.broadcasted_iota(jnp.int32, (TI, K), 1)
    asc = (iota3 & K) != 0
    d = K // 2
    while d >= 1:
        key, idx, pay = ce(key, idx, pay, iota3, d, asc)
        d //= 2

    o_ref[0] = pay


def kernel(x, W, b):
    B, L, D = x.shape
    N = D
    K = int((1.0 - 0.75) * L)
    TI = 256 if D % 256 == 0 else D
    grid = (B, D // TI)
    body = functools.partial(_body, N=N, K=K)
    out = pl.pallas_call(
        body,
        grid=grid,
        in_specs=[
            pl.BlockSpec((1, L, TI), lambda bb, i: (bb, 0, i)),
            pl.BlockSpec((D, L), lambda bb, i: (0, 0)),
            pl.BlockSpec((1, D), lambda bb, i: (0, 0)),
            pl.BlockSpec((1, TI, D), lambda bb, i: (bb, i, 0)),
        ],
        out_specs=pl.BlockSpec((1, TI, K), lambda bb, i: (bb, i, 0)),
        out_shape=jax.ShapeDtypeStruct((B, D, K), jnp.float32),
        compiler_params=pltpu.CompilerParams(
            dimension_semantics=("parallel", "parallel"),
        ),
    )(x, W, b.reshape(1, D), x)
    return jnp.transpose(out, (0, 2, 1))
